# baseline (speedup 1.0000x reference)
import functools

import jax
import jax.numpy as jnp
from jax import lax
from jax.experimental import pallas as pl
from jax.experimental.pallas import tpu as pltpu
from jax.experimental.pallas import tpu_sc as plsc

N_CLS = 1000
N_CTX = 16
D = 768
SEQ = 77
SUF = SEQ - 1 - N_CTX
LANES = 16
NJ = D // LANES

SUF_A = 32
SUF_B = SUF - SUF_A
P1 = 48
P2R = SEQ - P1

SC_CLS = 256
TC_BLK = 12


def _sc_concat(init, token_prefix, token_suffix, sc_cls):
    info = plsc.get_sparse_core_info()
    NC, NS = info.num_cores, info.num_subcores
    NW = NC * NS
    per_w = sc_cls // NW
    npairs = per_w // 2
    assert per_w % 2 == 0

    mesh = plsc.VectorSubcoreMesh(core_axis_name="c", subcore_axis_name="s")

    @functools.partial(
        pl.kernel,
        mesh=mesh,
        out_type=jax.ShapeDtypeStruct((N_CLS, SEQ, D), jnp.float32),
        scratch_types=[
            pltpu.VMEM((P1, D), jnp.float32),
            pltpu.VMEM((P1, D), jnp.float32),
            pltpu.VMEM((SUF_B, D), jnp.float32),
            pltpu.VMEM((P2R, D), jnp.float32),
            pltpu.VMEM((1, D), jnp.float32),
            pltpu.VMEM((1, D), jnp.float32),
            pltpu.VMEM((1, D), jnp.float32),
            pltpu.VMEM((1, D), jnp.float32),
            pltpu.SemaphoreType.DMA,
            pltpu.SemaphoreType.DMA,
            pltpu.SemaphoreType.DMA,
            pltpu.SemaphoreType.DMA,
            pltpu.SemaphoreType.DMA,
            pltpu.SemaphoreType.DMA,
            pltpu.SemaphoreType.DMA,
            pltpu.SemaphoreType.DMA,
        ],
    )
    def k(ctx_hbm, pre_hbm, suf_hbm, out_hbm,
          a0, a1, sb, p2, ctx15, pre_a, pre_b, tbuf,
          s_ia0, s_ia1, s_isb, s_pa, s_pb, s_sa0, s_sa1, s_sp2):
        cid = lax.axis_index("c")
        sid = lax.axis_index("s")
        wid = sid * NC + cid
        lo = per_w * wid

        def clamp(c):
            return jnp.minimum(c, N_CLS - 1)

        def vrow(dst, dr, src, sr):
            for j in range(NJ):
                dst[dr, pl.ds(j * LANES, LANES)] = src[sr, pl.ds(j * LANES, LANES)]

        def in_a(c, a, sem):
            return pltpu.make_async_copy(
                suf_hbm.at[c, pl.ds(0, SUF_A)], a.at[pl.ds(N_CTX, SUF_A)], sem)

        def in_sb(c, sem):
            return pltpu.make_async_copy(
                suf_hbm.at[c, pl.ds(SUF_A, SUF_B)], sb, sem)

        def in_pre(c, buf, sem):
            return pltpu.make_async_copy(pre_hbm.at[c], buf, sem)

        def st_a(c, a, sem):
            return pltpu.make_async_copy(a, out_hbm.at[c, pl.ds(0, P1)], sem)

        def st_p2(c, sem):
            return pltpu.make_async_copy(p2, out_hbm.at[c, pl.ds(P1, P2R)], sem)

        def shift_a(a):
            vrow(tbuf, 0, a, P1 - 1)
            for r in range(SUF_A - 2, -1, -1):
                vrow(a, 1 + N_CTX + r, a, N_CTX + r)
            vrow(a, N_CTX, ctx15, 0)

        pltpu.sync_copy(ctx_hbm, a0.at[pl.ds(0, N_CTX)])
        vrow(ctx15, 0, a0, N_CTX - 1)
        for r in range(N_CTX - 2, -1, -1):
            vrow(a0, 1 + r, a0, r)
        for r in range(1, N_CTX):
            vrow(a1, r, a0, r)

        in_a(lo, a0, s_ia0).start()
        in_a(lo + 1, a1, s_ia1).start()
        in_sb(lo, s_isb).start()
        in_pre(lo, pre_a, s_pa).start()
        in_pre(lo + 1, pre_b, s_pb).start()

        def pair(p, carry):
            c0 = lo + 2 * p
            c1 = c0 + 1

            @pl.when(p > 0)
            def _():
                st_a(c0, a1, s_sa1).wait()
                in_a(c1, a1, s_ia1).start()

            in_a(c0, a0, s_ia0).wait()
            shift_a(a0)
            in_pre(c0, pre_a, s_pa).wait()
            vrow(a0, 0, pre_a, 0)
            st_a(c0, a0, s_sa0).start()
            in_pre(clamp(c0 + 2), pre_a, s_pa).start()

            @pl.when(p > 0)
            def _():
                st_p2(c0, s_sp2).wait()
            vrow(p2, 0, tbuf, 0)
            in_sb(c0, s_isb).wait()
            for q in range(SUF_B):
                vrow(p2, 1 + q, sb, q)
            st_p2(c0, s_sp2).start()
            in_sb(c1, s_isb).start()

            in_a(c1, a1, s_ia1).wait()
            shift_a(a1)
            in_pre(c1, pre_b, s_pb).wait()
            vrow(a1, 0, pre_b, 0)
            st_a(c1, a1, s_sa1).start()
            in_pre(clamp(c1 + 2), pre_b, s_pb).start()

            st_a(c0, a0, s_sa0).wait()
            in_a(clamp(c0 + 2), a0, s_ia0).start()

            st_p2(c0, s_sp2).wait()
            vrow(p2, 0, tbuf, 0)
            in_sb(c1, s_isb).wait()
            for q in range(SUF_B):
                vrow(p2, 1 + q, sb, q)
            st_p2(c1, s_sp2).start()
            in_sb(clamp(c1 + 1), s_isb).start()
            return carry

        lax.fori_loop(0, npairs, pair, 0)

        st_a(0, a1, s_sa1).wait()
        st_p2(0, s_sp2).wait()
        in_a(0, a0, s_ia0).wait()
        in_sb(0, s_isb).wait()
        in_pre(0, pre_a, s_pa).wait()
        in_pre(0, pre_b, s_pb).wait()

    return k(init, token_prefix, token_suffix)


def _tc_fill_body(first, blk, nblk,
                  ctx_hbm, pre_hbm, suf_hbm, donated_hbm, out_hbm,
                  ctx_v, pre0, pre1, suf0, suf1, out0, out1,
                  s_i0, s_i1, s_p0, s_p1, s_o0, s_o1):
    npairs = nblk // 2

    def in_suf(b, buf, sem):
        return pltpu.make_async_copy(
            suf_hbm.at[pl.ds(first + b * blk, blk)], buf, sem)

    def in_pre(b, buf, sem):
        return pltpu.make_async_copy(
            pre_hbm.at[pl.ds(first + b * blk, blk)], buf, sem)

    def st_out(b, buf, sem):
        return pltpu.make_async_copy(
            buf, out_hbm.at[pl.ds(first + b * blk, blk)], sem)

    def clampb(b):
        return jnp.minimum(b, nblk - 1)

    pltpu.sync_copy(ctx_hbm, ctx_v)
    ctx_block = jnp.broadcast_to(ctx_v[...][None, :, :], (blk, N_CTX, D))
    out0[:, 1:1 + N_CTX, :] = ctx_block
    out1[:, 1:1 + N_CTX, :] = ctx_block

    in_suf(0, suf0, s_i0).start()
    in_suf(1, suf1, s_i1).start()
    in_pre(0, pre0, s_p0).start()
    in_pre(1, pre1, s_p1).start()

    def pair(p, carry):
        b0 = 2 * p
        b1 = b0 + 1

        @pl.when(p > 0)
        def _():
            st_out(b0, out0, s_o0).wait()
        in_suf(b0, suf0, s_i0).wait()
        in_pre(b0, pre0, s_p0).wait()
        out0[:, 0:1, :] = pre0[...]
        out0[:, 1 + N_CTX:, :] = suf0[...]
        st_out(b0, out0, s_o0).start()
        in_suf(clampb(b0 + 2), suf0, s_i0).start()
        in_pre(clampb(b0 + 2), pre0, s_p0).start()

        @pl.when(p > 0)
        def _():
            st_out(b1, out1, s_o1).wait()
        in_suf(b1, suf1, s_i1).wait()
        in_pre(b1, pre1, s_p1).wait()
        out1[:, 0:1, :] = pre1[...]
        out1[:, 1 + N_CTX:, :] = suf1[...]
        st_out(b1, out1, s_o1).start()
        in_suf(clampb(b1 + 2), suf1, s_i1).start()
        in_pre(clampb(b1 + 2), pre1, s_p1).start()
        return carry

    lax.fori_loop(0, npairs, pair, 0)

    st_out(0, out0, s_o0).wait()
    st_out(0, out1, s_o1).wait()
    in_suf(0, suf0, s_i0).wait()
    in_suf(0, suf1, s_i1).wait()
    in_pre(0, pre0, s_p0).wait()
    in_pre(0, pre1, s_p1).wait()


def _tc_fill(init, token_prefix, token_suffix, donated, first, blk):
    n = N_CLS - first
    nblk = n // blk
    assert n % blk == 0 and nblk % 2 == 0
    body = functools.partial(_tc_fill_body, first, blk, nblk)
    return pl.pallas_call(
        body,
        in_specs=[
            pl.BlockSpec(memory_space=pl.ANY),
            pl.BlockSpec(memory_space=pl.ANY),
            pl.BlockSpec(memory_space=pl.ANY),
            pl.BlockSpec(memory_space=pl.ANY),
        ],
        out_specs=pl.BlockSpec(memory_space=pl.ANY),
        out_shape=jax.ShapeDtypeStruct((N_CLS, SEQ, D), jnp.float32),
        input_output_aliases={3: 0},
        scratch_shapes=[
            pltpu.VMEM((N_CTX, D), jnp.float32),
            pltpu.VMEM((blk, 1, D), jnp.float32),
            pltpu.VMEM((blk, 1, D), jnp.float32),
            pltpu.VMEM((blk, SUF, D), jnp.float32),
            pltpu.VMEM((blk, SUF, D), jnp.float32),
            pltpu.VMEM((blk, SEQ, D), jnp.float32),
            pltpu.VMEM((blk, SEQ, D), jnp.float32),
            pltpu.SemaphoreType.DMA,
            pltpu.SemaphoreType.DMA,
            pltpu.SemaphoreType.DMA,
            pltpu.SemaphoreType.DMA,
            pltpu.SemaphoreType.DMA,
            pltpu.SemaphoreType.DMA,
        ],
        compiler_params=pltpu.CompilerParams(
            vmem_limit_bytes=100 * 1024 * 1024),
    )(init, token_prefix, token_suffix, donated)


def kernel(init, token_prefix, token_suffix):
    sc_out = _sc_concat(init, token_prefix, token_suffix, SC_CLS)
    return _tc_fill(init, token_prefix, token_suffix, sc_out, SC_CLS, TC_BLK)
